# Initial kernel scaffold; baseline (speedup 1.0000x reference)
import jax
import jax.numpy as jnp
from jax.experimental import pallas as pl


def kernel(input_ids, embed_weight):
    raise NotImplementedError("write your pallas kernel here")


# speedup vs baseline: 4.3625x; 4.3625x over previous
import functools

import jax
import jax.numpy as jnp
from jax import lax
from jax.experimental import pallas as pl
from jax.experimental.pallas import tpu as pltpu
from jax.experimental.pallas import tpu_sc as plsc

_VOCAB = 100000
_D = 64
_MULT = 8.0

_NC = 2
_NS = 16
_NW = _NC * _NS

_C = 128
_NBUF = 4


def _scale_body(w_ref, o_ref):
    o_ref[...] = w_ref[...] * _MULT


def _scale_table(w):
    rows_per_block = 4000
    grid = _VOCAB // rows_per_block
    return pl.pallas_call(
        _scale_body,
        out_shape=jax.ShapeDtypeStruct((_VOCAB, _D), jnp.float32),
        grid=(grid,),
        in_specs=[pl.BlockSpec((rows_per_block, _D), lambda i: (i, 0))],
        out_specs=pl.BlockSpec((rows_per_block, _D), lambda i: (i, 0)),
    )(w)


def _make_gather(total_rows):
    assert total_rows % (_NW * _C) == 0
    per_w = total_rows // _NW
    groups = per_w // _C
    main = groups - _NBUF
    assert main % _NBUF == 0
    mesh = plsc.VectorSubcoreMesh(core_axis_name="c", subcore_axis_name="s")

    @functools.partial(
        pl.kernel,
        out_type=jax.ShapeDtypeStruct((total_rows, _D), jnp.float32),
        mesh=mesh,
        scratch_types=(
            [pltpu.VMEM((groups, _C), jnp.int32)]
            + [pltpu.VMEM((_C, _D), jnp.float32) for _ in range(_NBUF)]
            + [pltpu.SemaphoreType.DMA for _ in range(2 * _NBUF)]
        ),
        compiler_params=pltpu.CompilerParams(use_tc_tiling_on_sc=False),
    )
    def gather_kernel(table_hbm, idx_hbm, out_hbm, idx_v, *rest):
        bufs = rest[:_NBUF]
        gsem = rest[_NBUF:2 * _NBUF]
        osem = rest[2 * _NBUF:]
        wid = lax.axis_index("s") * _NC + lax.axis_index("c")
        base = wid * per_w

        pltpu.sync_copy(idx_hbm.at[wid], idx_v)

        def g_start(b, g):
            pltpu.make_async_copy(
                table_hbm.at[idx_v.at[g]], bufs[b], gsem[b]).start()

        def g_wait(b):
            pltpu.make_async_copy(
                table_hbm.at[idx_v.at[0]], bufs[b], gsem[b]).wait()

        def o_start(b, g):
            pltpu.make_async_copy(
                bufs[b], out_hbm.at[pl.ds(base + g * _C, _C)], osem[b]).start()

        def o_wait(b):
            pltpu.make_async_copy(
                bufs[b], out_hbm.at[pl.ds(base, _C)], osem[b]).wait()

        for b in range(_NBUF):
            g_start(b, b)

        def step(go, carry):
            for b in range(_NBUF):
                g = go * _NBUF + b
                g_wait(b)
                o_start(b, g)
                o_wait(b)
                g_start(b, g + _NBUF)
            return carry

        lax.fori_loop(0, main // _NBUF, step, 0)

        for b in range(_NBUF):
            g = main + b
            g_wait(b)
            o_start(b, g)
        for b in range(_NBUF):
            o_wait(b)

    return gather_kernel


def kernel(input_ids, embed_weight):
    batch, seq = input_ids.shape
    total = batch * seq
    idx = input_ids.reshape(_NW, total // (_NW * _C), _C).astype(jnp.int32)
    table = _scale_table(embed_weight)
    out = _make_gather(total)(table, idx)
    return out.reshape(batch, seq, _D)
